# baseline (speedup 1.0000x reference)
import jax
import jax.numpy as jnp
from jax import lax
from jax.experimental import pallas as pl
from jax.experimental.pallas import tpu as pltpu

_B = 4096
_IN = 512
_LAT = 64
_K = 15
_BIG = 3.0e38

_RB1 = 512
_RB2 = 256
_RBS = 512
_NB1 = _B // _RB1
_NB2 = _B // _RB2
_NBS = _B // _RBS
_TRI = tuple((i, j) for i in range(_NBS) for j in range(i, _NBS))
_TRI_I = tuple(i for i, _ in _TRI)
_TRI_J = tuple(j for _, j in _TRI)

_NT = (((1,), (1,)), ((), ()))


def _mlp_body(x_ref, ew1, eb1, ew2, eb2, ew3, eb3, dw1, db1, dw2, db2, dw3,
              db3, z_ref, rec_ref, xsq_ref, zsq_ref):
    f32 = jnp.float32
    xb = x_ref[...]
    h = jnp.maximum(jnp.dot(xb, ew1[...], preferred_element_type=f32)
                    + eb1[...][None, :], 0.0)
    h = jnp.maximum(jnp.dot(h, ew2[...], preferred_element_type=f32)
                    + eb2[...][None, :], 0.0)
    z = jnp.dot(h, ew3[...], preferred_element_type=f32) + eb3[...][None, :]
    z_ref[...] = z
    g = jnp.maximum(jnp.dot(z, dw1[...], preferred_element_type=f32)
                    + db1[...][None, :], 0.0)
    g = jnp.maximum(jnp.dot(g, dw2[...], preferred_element_type=f32)
                    + db2[...][None, :], 0.0)
    xr = jnp.dot(g, dw3[...], preferred_element_type=f32) + db3[...][None, :]
    d = xr - xb
    part = jnp.sum((d * d).reshape(_RB1, _IN // 128, 128), axis=(0, 1))
    rec_ref[...] = part.reshape(1, 1, 128)
    xsq = jnp.sum(xb * xb, axis=1, keepdims=True)
    xsq_ref[...] = jnp.swapaxes(xsq, 0, 1)
    zsq = jnp.sum(z * z, axis=1, keepdims=True)
    zsq_ref[...] = jnp.swapaxes(zsq, 0, 1)


def _dist_body(xb_ref, xf_ref, xsqb_ref, xsqr_ref, trow_ref):
    f32 = jnp.float32
    i = pl.program_id(0)
    xb = xb_ref[...]
    xf = xf_ref[...]
    s = lax.dot_general(xb, xf, _NT, preferred_element_type=f32)
    xbsq = jnp.swapaxes(xsqb_ref[...], 0, 1)
    xfsq = xsqr_ref[...]
    sq = jnp.maximum(xbsq + xfsq - 2.0 * s, 0.0)
    col = lax.broadcasted_iota(jnp.int32, (_RB2, _B), 1)
    rowg = lax.broadcasted_iota(jnp.int32, (_RB2, _B), 0) + i * _RB2
    sq = jnp.where(col == rowg, _BIG, sq)
    cat_rows = []
    for r in range(_RB2 // 8):
        lv = [jnp.full((8, 128), _BIG, f32) for _ in range(5)]
        for a in range(_B // 128):
            cur = sq[r * 8:(r + 1) * 8, a * 128:(a + 1) * 128]
            for l in range(5):
                lo = jnp.minimum(lv[l], cur)
                cur = jnp.maximum(lv[l], cur)
                lv[l] = lo
        cat_rows.append(jnp.concatenate(lv, axis=1))
    cat = jnp.concatenate(cat_rows, axis=0)
    prev = jnp.full((_RB2, 1), -1.0, f32)
    for _ in range(_K):
        cand = jnp.where(cat > prev, cat, _BIG)
        prev = jnp.min(cand, axis=1, keepdims=True)
    trow_ref[...] = jnp.swapaxes(prev, 0, 1)


def _stats_body(xi_ref, xj_ref, xsqi_ref, xsqj_ref, ti_ref, tj_ref,
                zi_ref, zj_ref, zsqi_ref, zsqj_ref, out_ref):
    f32 = jnp.float32
    p = pl.program_id(0)
    bi = _tri_i(p)
    bj = _tri_j(p)
    xi = xi_ref[...]
    xj = xj_ref[...]
    sx = lax.dot_general(xi, xj, _NT, preferred_element_type=f32)
    xisq = jnp.swapaxes(xsqi_ref[...], 0, 1)
    xjsq = xsqj_ref[...]
    dsq = jnp.maximum(xisq + xjsq - 2.0 * sx, 0.0)
    rowg = lax.broadcasted_iota(jnp.int32, (_RBS, _RBS), 0) + bi * _RBS
    colg = lax.broadcasted_iota(jnp.int32, (_RBS, _RBS), 1) + bj * _RBS
    dsq = jnp.where(rowg == colg, _BIG, dsq)
    zi = zi_ref[...]
    zj = zj_ref[...]
    s = lax.dot_general(zi, zj, _NT, preferred_element_type=f32)
    zisq = jnp.swapaxes(zsqi_ref[...], 0, 1)
    zjsq = zsqj_ref[...]
    zsq = jnp.maximum(zisq + zjsq - 2.0 * s, 0.0)
    ti = jnp.swapaxes(ti_ref[...], 0, 1) * (1.0 + 1e-06)
    tj = tj_ref[...] * (1.0 + 1e-06)
    mask = (dsq <= ti) | (dsq <= tj)
    u2 = jnp.where(mask, dsq, 0.0)
    v2 = jnp.where(mask, zsq, 0.0)
    cnt = jnp.sum(mask.astype(f32))
    xmax = jnp.max(u2)
    zmax = jnp.max(v2)
    sxx = jnp.sum(u2)
    szz = jnp.sum(v2)
    sxz = jnp.sum(jnp.where(mask, jnp.sqrt(dsq * zsq), 0.0))
    lane = lax.broadcasted_iota(jnp.int32, (1, 1, 128), 2)
    vec = jnp.where(lane == 0, cnt,
          jnp.where(lane == 1, xmax,
          jnp.where(lane == 2, zmax,
          jnp.where(lane == 3, sxx,
          jnp.where(lane == 4, szz,
          jnp.where(lane == 5, sxz, 0.0))))))
    out_ref[...] = vec


def _tri_i(p):
    c = jnp.float32(2 * _NBS + 1)
    pf = jnp.asarray(p, jnp.float32)
    return jnp.floor((c - jnp.sqrt(c * c - 8.0 * pf)) * 0.5).astype(jnp.int32)


def _tri_j(p):
    i = _tri_i(p)
    off = _NBS * i - (i * (i - 1)) // 2
    return i + (p - off)


def kernel(x, e_w1, e_b1, e_w2, e_b2, e_w3, e_b3,
           d_w1, d_b1, d_w2, d_b2, d_w3, d_b3):
    f32 = jnp.float32
    full2 = lambda i: (0, 0)
    full1 = lambda i: (0,)

    z, recp, xsqrow, zsqrow = pl.pallas_call(
        _mlp_body,
        grid=(_NB1,),
        in_specs=[
            pl.BlockSpec((_RB1, _IN), lambda i: (i, 0)),
            pl.BlockSpec((_IN, 1024), full2), pl.BlockSpec((1024,), full1),
            pl.BlockSpec((1024, 512), full2), pl.BlockSpec((512,), full1),
            pl.BlockSpec((512, _LAT), full2), pl.BlockSpec((_LAT,), full1),
            pl.BlockSpec((_LAT, 512), full2), pl.BlockSpec((512,), full1),
            pl.BlockSpec((512, 1024), full2), pl.BlockSpec((1024,), full1),
            pl.BlockSpec((1024, _IN), full2), pl.BlockSpec((_IN,), full1),
        ],
        out_specs=[
            pl.BlockSpec((_RB1, _LAT), lambda i: (i, 0)),
            pl.BlockSpec((1, 1, 128), lambda i: (i, 0, 0)),
            pl.BlockSpec((1, _RB1), lambda i: (0, i)),
            pl.BlockSpec((1, _RB1), lambda i: (0, i)),
        ],
        out_shape=[
            jax.ShapeDtypeStruct((_B, _LAT), f32),
            jax.ShapeDtypeStruct((_NB1, 1, 128), f32),
            jax.ShapeDtypeStruct((1, _B), f32),
            jax.ShapeDtypeStruct((1, _B), f32),
        ],
    )(x, e_w1, e_b1, e_w2, e_b2, e_w3, e_b3,
      d_w1, d_b1, d_w2, d_b2, d_w3, d_b3)

    tsqrow = pl.pallas_call(
        _dist_body,
        grid=(_NB2,),
        in_specs=[
            pl.BlockSpec((_RB2, _IN), lambda i: (i, 0)),
            pl.BlockSpec((_B, _IN), full2),
            pl.BlockSpec((1, _RB2), lambda i: (0, i)),
            pl.BlockSpec((1, _B), full2),
        ],
        out_specs=pl.BlockSpec((1, _RB2), lambda i: (0, i)),
        out_shape=jax.ShapeDtypeStruct((1, _B), f32),
    )(x, x, xsqrow, xsqrow)

    stats = pl.pallas_call(
        _stats_body,
        grid=(len(_TRI),),
        in_specs=[
            pl.BlockSpec((_RBS, _IN), lambda p: (_tri_i(p), 0)),
            pl.BlockSpec((_RBS, _IN), lambda p: (_tri_j(p), 0)),
            pl.BlockSpec((1, _RBS), lambda p: (0, _tri_i(p))),
            pl.BlockSpec((1, _RBS), lambda p: (0, _tri_j(p))),
            pl.BlockSpec((1, _RBS), lambda p: (0, _tri_i(p))),
            pl.BlockSpec((1, _RBS), lambda p: (0, _tri_j(p))),
            pl.BlockSpec((_RBS, _LAT), lambda p: (_tri_i(p), 0)),
            pl.BlockSpec((_RBS, _LAT), lambda p: (_tri_j(p), 0)),
            pl.BlockSpec((1, _RBS), lambda p: (0, _tri_i(p))),
            pl.BlockSpec((1, _RBS), lambda p: (0, _tri_j(p))),
        ],
        out_specs=pl.BlockSpec((1, 1, 128), lambda p: (p, 0, 0)),
        out_shape=jax.ShapeDtypeStruct((len(_TRI), 1, 128), f32),
    )(x, x, xsqrow, xsqrow, tsqrow, tsqrow, z, z, zsqrow, zsqrow)

    P = stats.reshape(len(_TRI), 128)
    w = jnp.asarray([1.0 if i == j else 2.0 for (i, j) in _TRI], f32)
    cnt = jnp.sum(P[:, 0] * w)
    xm = jnp.sqrt(jnp.max(P[:, 1])) + 1e-08
    zm = jnp.sqrt(jnp.max(P[:, 2])) + 1e-08
    sxx = jnp.sum(P[:, 3] * w)
    szz = jnp.sum(P[:, 4] * w)
    sxz = jnp.sum(P[:, 5] * w)
    rec_loss = jnp.sum(recp) / (_B * _IN)
    knn_loss = (szz / (zm * zm) - 2.0 * sxz / (xm * zm) + sxx / (xm * xm)) / cnt
    total = rec_loss + knn_loss
    return total, z, rec_loss, knn_loss


# speedup vs baseline: 1.1688x; 1.1688x over previous
import jax
import jax.numpy as jnp
from jax import lax
from jax.experimental import pallas as pl

_B = 4096
_IN = 512
_LAT = 64
_K = 15
_BIG = 3.0e38

_RB1 = 512
_RB2 = 512
_RBS = 1024
_NB1 = _B // _RB1
_NB2 = _B // _RB2
_NBS = _B // _RBS
_TRI = tuple((i, j) for i in range(_NBS) for j in range(i, _NBS))
_TRI_I = tuple(i for i, _ in _TRI)
_TRI_J = tuple(j for _, j in _TRI)

_NT = (((1,), (1,)), ((), ()))


def _mlp_body(x_ref, ew1, eb1, ew2, eb2, ew3, eb3, dw1, db1, dw2, db2, dw3,
              db3, z_ref, rec_ref, xsq_ref, zsq_ref):
    f32 = jnp.float32
    xb = x_ref[...]
    h = jnp.maximum(jnp.dot(xb, ew1[...], preferred_element_type=f32)
                    + eb1[...][None, :], 0.0)
    h = jnp.maximum(jnp.dot(h, ew2[...], preferred_element_type=f32)
                    + eb2[...][None, :], 0.0)
    z = jnp.dot(h, ew3[...], preferred_element_type=f32) + eb3[...][None, :]
    z_ref[...] = z
    bf16 = jnp.bfloat16
    g = jnp.maximum(jnp.dot(z.astype(bf16), dw1[...],
                            preferred_element_type=f32)
                    + db1[...][None, :], 0.0)
    g = jnp.maximum(jnp.dot(g.astype(bf16), dw2[...],
                            preferred_element_type=f32)
                    + db2[...][None, :], 0.0)
    xr = (jnp.dot(g.astype(bf16), dw3[...], preferred_element_type=f32)
          + db3[...][None, :])
    d = xr - xb
    part = jnp.sum((d * d).reshape(_RB1, _IN // 128, 128), axis=(0, 1))
    rec_ref[...] = part.reshape(1, 1, 128)
    xsq = jnp.sum(xb * xb, axis=1, keepdims=True)
    xsq_ref[...] = jnp.swapaxes(xsq, 0, 1)
    zsq = jnp.sum(z * z, axis=1, keepdims=True)
    zsq_ref[...] = jnp.swapaxes(zsq, 0, 1)


def _dist_body(xb_ref, xf_ref, xsqb_ref, xsqr_ref, d_ref, trow_ref):
    f32 = jnp.float32
    i = pl.program_id(0)
    xb = xb_ref[...]
    xf = xf_ref[...]
    s = lax.dot_general(xb, xf, _NT, preferred_element_type=f32)
    xbsq = jnp.swapaxes(xsqb_ref[...], 0, 1)
    xfsq = xsqr_ref[...]
    sq = jnp.maximum(xbsq + xfsq - 2.0 * s, 0.0)
    col = lax.broadcasted_iota(jnp.int32, (_RB2, _B), 1)
    rowg = lax.broadcasted_iota(jnp.int32, (_RB2, _B), 0) + i * _RB2
    sq = jnp.where(col == rowg, _BIG, sq)
    d_ref[...] = sq
    cat_rows = []
    for r in range(_RB2 // 8):
        lv = [jnp.full((8, 128), _BIG, f32) for _ in range(5)]
        for a in range(_B // 128):
            cur = sq[r * 8:(r + 1) * 8, a * 128:(a + 1) * 128]
            for l in range(5):
                lo = jnp.minimum(lv[l], cur)
                cur = jnp.maximum(lv[l], cur)
                lv[l] = lo
        cat_rows.append(jnp.concatenate(lv, axis=1))
    cat = jnp.concatenate(cat_rows, axis=0)
    prev = jnp.full((_RB2, 1), -1.0, f32)
    for _ in range(_K):
        cand = jnp.where(cat > prev, cat, _BIG)
        prev = jnp.min(cand, axis=1, keepdims=True)
    trow_ref[...] = jnp.swapaxes(prev, 0, 1)


def _stats_body(d_ref, ti_ref, tj_ref, zi_ref, zj_ref, zsqi_ref, zsqj_ref,
                out_ref):
    f32 = jnp.float32
    dsq = d_ref[...]
    zi = zi_ref[...]
    zj = zj_ref[...]
    s = lax.dot_general(zi, zj, _NT, preferred_element_type=f32)
    zisq = jnp.swapaxes(zsqi_ref[...], 0, 1)
    zjsq = zsqj_ref[...]
    zsq = jnp.maximum(zisq + zjsq - 2.0 * s, 0.0)
    ti = jnp.swapaxes(ti_ref[...], 0, 1)
    tj = tj_ref[...]
    mask = (dsq <= ti) | (dsq <= tj)
    u2 = jnp.where(mask, dsq, 0.0)
    v2 = jnp.where(mask, zsq, 0.0)
    cnt = jnp.sum(mask.astype(f32))
    xmax = jnp.max(u2)
    zmax = jnp.max(v2)
    sxx = jnp.sum(u2)
    szz = jnp.sum(v2)
    sxz = jnp.sum(jnp.where(mask, jnp.sqrt(dsq * zsq), 0.0))
    lane = lax.broadcasted_iota(jnp.int32, (1, 1, 128), 2)
    vec = jnp.where(lane == 0, cnt,
          jnp.where(lane == 1, xmax,
          jnp.where(lane == 2, zmax,
          jnp.where(lane == 3, sxx,
          jnp.where(lane == 4, szz,
          jnp.where(lane == 5, sxz, 0.0))))))
    out_ref[...] = vec


def _tri_i(p):
    c = jnp.float32(2 * _NBS + 1)
    pf = jnp.asarray(p, jnp.float32)
    return jnp.floor((c - jnp.sqrt(c * c - 8.0 * pf)) * 0.5).astype(jnp.int32)


def _tri_j(p):
    i = _tri_i(p)
    off = _NBS * i - (i * (i - 1)) // 2
    return i + (p - off)


def kernel(x, e_w1, e_b1, e_w2, e_b2, e_w3, e_b3,
           d_w1, d_b1, d_w2, d_b2, d_w3, d_b3):
    f32 = jnp.float32
    full2 = lambda i: (0, 0)
    full1 = lambda i: (0,)

    z, recp, xsqrow, zsqrow = pl.pallas_call(
        _mlp_body,
        grid=(_NB1,),
        in_specs=[
            pl.BlockSpec((_RB1, _IN), lambda i: (i, 0)),
            pl.BlockSpec((_IN, 1024), full2), pl.BlockSpec((1024,), full1),
            pl.BlockSpec((1024, 512), full2), pl.BlockSpec((512,), full1),
            pl.BlockSpec((512, _LAT), full2), pl.BlockSpec((_LAT,), full1),
            pl.BlockSpec((_LAT, 512), full2), pl.BlockSpec((512,), full1),
            pl.BlockSpec((512, 1024), full2), pl.BlockSpec((1024,), full1),
            pl.BlockSpec((1024, _IN), full2), pl.BlockSpec((_IN,), full1),
        ],
        out_specs=[
            pl.BlockSpec((_RB1, _LAT), lambda i: (i, 0)),
            pl.BlockSpec((1, 1, 128), lambda i: (i, 0, 0)),
            pl.BlockSpec((1, _RB1), lambda i: (0, i)),
            pl.BlockSpec((1, _RB1), lambda i: (0, i)),
        ],
        out_shape=[
            jax.ShapeDtypeStruct((_B, _LAT), f32),
            jax.ShapeDtypeStruct((_NB1, 1, 128), f32),
            jax.ShapeDtypeStruct((1, _B), f32),
            jax.ShapeDtypeStruct((1, _B), f32),
        ],
    )(x, e_w1, e_b1, e_w2, e_b2, e_w3, e_b3,
      d_w1.astype(jnp.bfloat16), d_b1, d_w2.astype(jnp.bfloat16), d_b2,
      d_w3.astype(jnp.bfloat16), d_b3)

    dmat, tsqrow = pl.pallas_call(
        _dist_body,
        grid=(_NB2,),
        in_specs=[
            pl.BlockSpec((_RB2, _IN), lambda i: (i, 0)),
            pl.BlockSpec((_B, _IN), full2),
            pl.BlockSpec((1, _RB2), lambda i: (0, i)),
            pl.BlockSpec((1, _B), full2),
        ],
        out_specs=[
            pl.BlockSpec((_RB2, _B), lambda i: (i, 0)),
            pl.BlockSpec((1, _RB2), lambda i: (0, i)),
        ],
        out_shape=[
            jax.ShapeDtypeStruct((_B, _B), f32),
            jax.ShapeDtypeStruct((1, _B), f32),
        ],
    )(x, x, xsqrow, xsqrow)

    stats = pl.pallas_call(
        _stats_body,
        grid=(len(_TRI),),
        in_specs=[
            pl.BlockSpec((_RBS, _RBS), lambda p: (_tri_i(p), _tri_j(p))),
            pl.BlockSpec((1, _RBS), lambda p: (0, _tri_i(p))),
            pl.BlockSpec((1, _RBS), lambda p: (0, _tri_j(p))),
            pl.BlockSpec((_RBS, _LAT), lambda p: (_tri_i(p), 0)),
            pl.BlockSpec((_RBS, _LAT), lambda p: (_tri_j(p), 0)),
            pl.BlockSpec((1, _RBS), lambda p: (0, _tri_i(p))),
            pl.BlockSpec((1, _RBS), lambda p: (0, _tri_j(p))),
        ],
        out_specs=pl.BlockSpec((1, 1, 128), lambda p: (p, 0, 0)),
        out_shape=jax.ShapeDtypeStruct((len(_TRI), 1, 128), f32),
    )(dmat, tsqrow, tsqrow, z, z, zsqrow, zsqrow)

    P = stats.reshape(len(_TRI), 128)
    w = jnp.asarray([1.0 if i == j else 2.0 for (i, j) in _TRI], f32)
    cnt = jnp.sum(P[:, 0] * w)
    xm = jnp.sqrt(jnp.max(P[:, 1])) + 1e-08
    zm = jnp.sqrt(jnp.max(P[:, 2])) + 1e-08
    sxx = jnp.sum(P[:, 3] * w)
    szz = jnp.sum(P[:, 4] * w)
    sxz = jnp.sum(P[:, 5] * w)
    rec_loss = jnp.sum(recp) / (_B * _IN)
    knn_loss = (szz / (zm * zm) - 2.0 * sxz / (xm * zm) + sxx / (xm * xm)) / cnt
    total = rec_loss + knn_loss
    return total, z, rec_loss, knn_loss
